# baseline (speedup 1.0000x reference)
import jax
import jax.numpy as jnp
from jax import lax
from jax.experimental import pallas as pl
from jax.experimental.pallas import tpu as pltpu
from jax.experimental.pallas import tpu_sc as plsc

N = 50000
E = 800000
H = 64
HQ = 16
NF = N // 2

CHUNK = 512
GK = 1
NG = 10
NMB = 10
CPM = NG * GK
E_PAD = 16 * NMB * NG * GK * CHUNK
BE = E_PAD // 25

T_ROWS = 50176
ACC_ROWS = 50176
ZROWS = 64
M_PAD = 50048
ROWS_PER_SUB = M_PAD // 16
PAD_DST = M_PAD


def _seg_body(t4, sx, dx, m_out, spt, acc, sv, dv, rv0, rv1, zbuf,
              gsem0, gsem1, ssem0, ssem1):
    c = lax.axis_index("c")
    s = lax.axis_index("s")

    zv = jnp.zeros((16,), jnp.float32)

    def _zb(i, carry):
        zbuf[i, pl.ds(0, 16)] = zv
        return carry

    lax.fori_loop(0, ZROWS, _zb, 0)

    rv = (rv0, rv1)
    gsem = (gsem0, gsem1)
    ssem = (ssem0, ssem1)

    def _drain_scatters(B):
        for j in range(GK):
            pltpu.make_async_copy(
                rv[B].at[j], acc.at[dv.at[j]], ssem[B]
            ).wait()

    for q_loc in (0, 1):
        q = 2 * c + q_loc

        def _za(k, carry):
            pltpu.sync_copy(zbuf, acc.at[pl.ds(s * 3136 + k * ZROWS, ZROWS)])
            return carry

        lax.fori_loop(0, 49, _za, 0)
        pltpu.sync_copy(
            t4.at[pl.ds(s * 3136, 3136), q],
            spt.at[pl.ds(s * 3136, 3136)],
        )
        plsc.subcore_barrier()

        def _mega(mb, carry):
            @pl.when(mb > 0)
            def _():
                _drain_scatters(0)
                _drain_scatters(1)

            pltpu.sync_copy(sx.at[s, mb], sv)
            pltpu.sync_copy(dx.at[s, mb], dv)

            def _gpair(gp, inner):
                for B in (0, 1):
                    g = 2 * gp + B

                    @pl.when(g >= 2)
                    def _():
                        _drain_scatters(B)

                    descs = []
                    for j in range(GK):
                        descs.append(pltpu.async_copy(
                            spt.at[sv.at[g * GK + j]], rv[B].at[j], gsem[B]
                        ))
                    for d in descs:
                        d.wait()
                    for j in range(GK):
                        pltpu.async_copy(
                            rv[B].at[j], acc.at[dv.at[g * GK + j]], ssem[B],
                            add=True,
                        )
                return inner

            lax.fori_loop(0, NG // 2, _gpair, 0)
            return carry

        lax.fori_loop(0, NMB, _mega, 0)
        _drain_scatters(0)
        _drain_scatters(1)
        plsc.subcore_barrier()

        pltpu.sync_copy(
            acc.at[pl.ds(s * ROWS_PER_SUB, ROWS_PER_SUB)],
            m_out.at[pl.ds(s * ROWS_PER_SUB, ROWS_PER_SUB), q],
        )
        plsc.subcore_barrier()


def _segment_sum(t4, sx, dx):
    mesh = plsc.VectorSubcoreMesh(
        core_axis_name="c", subcore_axis_name="s", num_cores=2, num_subcores=16
    )
    return pl.kernel(
        _seg_body,
        out_type=jax.ShapeDtypeStruct((M_PAD, 4, HQ), jnp.float32),
        mesh=mesh,
        scratch_types=[
            pltpu.VMEM_SHARED((T_ROWS, HQ), jnp.float32),
            pltpu.VMEM_SHARED((ACC_ROWS, HQ), jnp.float32),
            pltpu.VMEM((CPM, CHUNK), jnp.int32),
            pltpu.VMEM((CPM, CHUNK), jnp.int32),
            pltpu.VMEM((GK, CHUNK, HQ), jnp.float32),
            pltpu.VMEM((GK, CHUNK, HQ), jnp.float32),
            pltpu.VMEM((ZROWS, HQ), jnp.float32),
            pltpu.SemaphoreType.DMA,
            pltpu.SemaphoreType.DMA,
            pltpu.SemaphoreType.DMA,
            pltpu.SemaphoreType.DMA,
        ],
        compiler_params=pltpu.CompilerParams(use_tc_tiling_on_sc=False),
    )(t4, sx, dx)


BRF = 1000


def _s1_body(xf_ref, wi2_ref, bi2_ref, wm2_ref, whr, whz, whn, eix_ref,
             h0f_ref, tf_ref, uhr_ref, uhz_ref, uhn_ref, sx_ref, dx_ref):
    i = pl.program_id(0)
    h0f = jnp.tanh(
        jnp.dot(xf_ref[...], wi2_ref[...], preferred_element_type=jnp.float32)
        + bi2_ref[...]
    )
    h0f_ref[...] = h0f

    def dot(a, b):
        return jnp.dot(a, b[...], preferred_element_type=jnp.float32)

    tf_ref[...] = dot(h0f, wm2_ref)
    uhr_ref[...] = dot(h0f, whr)
    uhz_ref[...] = dot(h0f, whz)
    uhn_ref[...] = dot(h0f, whn)

    eid = i * BE + lax.broadcasted_iota(jnp.int32, (1, BE), 1)
    valid = eid < E
    src = eix_ref[0:1, :]
    dst = eix_ref[1:2, :]
    sx_ref[...] = jnp.where(valid, src, 0).reshape(1, 1, BE)
    dx_ref[...] = jnp.where(valid, dst, PAD_DST).reshape(1, 1, BE)


def _stage1(xf, W2_init, b2_init, W2_msg, Wd_hr, Wd_hz, Wd_hn, edge_index):
    full = pl.BlockSpec((2 * H, 2 * H), lambda i: (0, 0))
    rows = pl.BlockSpec((BRF, 2 * H), lambda i: (i, 0))
    out = jax.ShapeDtypeStruct((NF, 2 * H), jnp.float32)
    return pl.pallas_call(
        _s1_body,
        grid=(25,),
        in_specs=[
            pl.BlockSpec((BRF, 26), lambda i: (i, 0)),
            pl.BlockSpec((26, 2 * H), lambda i: (0, 0)),
            pl.BlockSpec((1, 2 * H), lambda i: (0, 0)),
            full, full, full, full,
            pl.BlockSpec((2, BE), lambda i: (0, i)),
        ],
        out_specs=[
            rows, rows, rows, rows, rows,
            pl.BlockSpec((1, 1, BE), lambda i: (i, 0, 0)),
            pl.BlockSpec((1, 1, BE), lambda i: (i, 0, 0)),
        ],
        out_shape=[
            out,
            jax.ShapeDtypeStruct((T_ROWS // 2, 2 * H), jnp.float32),
            out, out, out,
            jax.ShapeDtypeStruct((25, 1, BE), jnp.int32),
            jax.ShapeDtypeStruct((25, 1, BE), jnp.int32),
        ],
    )(xf, W2_init, b2_init, W2_msg, Wd_hr, Wd_hz, Wd_hn, edge_index)


def _s3_body(mf_ref, h0f_ref, uhr_ref, uhz_ref, uhn_ref, wir, wiz, win, br,
             bz, bn, wg, wf, wfin, bfin, nn_ref, hg_ref, comb_ref):
    i = pl.program_id(0)
    mf = mf_ref[...]
    h0f = h0f_ref[...]

    def dot(a, b):
        return jnp.dot(a, b[...], preferred_element_type=jnp.float32)

    r = jax.nn.sigmoid(dot(mf, wir) + uhr_ref[...] + br[...])
    z = jax.nn.sigmoid(dot(mf, wiz) + uhz_ref[...] + bz[...])
    cand = jnp.tanh(dot(mf, win) + r * uhn_ref[...] + bn[...])
    h = (1.0 - z) * cand + z * h0f
    gate = jax.nn.sigmoid(dot(h, wg))
    feat = jnp.tanh(dot(h, wf))
    part = jnp.sum(gate * feat, axis=0, keepdims=True)

    @pl.when(i == 0)
    def _():
        hg_ref[...] = jnp.zeros_like(hg_ref)
        comb_ref[...] = jnp.zeros_like(comb_ref)

    hg_ref[...] += part

    @pl.when(i == pl.num_programs(0) - 1)
    def _():
        acc = hg_ref[...]
        hg64 = acc[:, 0:H] + acc[:, H:2 * H]
        cat = jnp.concatenate([hg64, nn_ref[...]], axis=1)
        comb_ref[...] = jnp.tanh(
            jnp.dot(cat, wfin[...], preferred_element_type=jnp.float32)
            + bfin[...]
        )


def _stage3(mf, h0f, uhr, uhz, uhn, nn, Wd_ir, Wd_iz, Wd_in, b2_r, b2_z,
            b2_n, Wd_gate, Wd_feat, W_final, b_final):
    full = pl.BlockSpec((2 * H, 2 * H), lambda i: (0, 0))
    vec2 = pl.BlockSpec((1, 2 * H), lambda i: (0, 0))
    vec = pl.BlockSpec((1, H), lambda i: (0, 0))
    rows = pl.BlockSpec((BRF, 2 * H), lambda i: (i, 0))
    return pl.pallas_call(
        _s3_body,
        grid=(25,),
        in_specs=[rows, rows, rows, rows, rows, full, full, full, vec2, vec2,
                  vec2, full, full, pl.BlockSpec((2 * H, H), lambda i: (0, 0)),
                  vec, vec],
        out_specs=[vec2, vec],
        out_shape=[
            jax.ShapeDtypeStruct((1, 2 * H), jnp.float32),
            jax.ShapeDtypeStruct((1, H), jnp.float32),
        ],
    )(mf, h0f, uhr, uhz, uhn, Wd_ir, Wd_iz, Wd_in, b2_r, b2_z, b2_n,
      Wd_gate, Wd_feat, W_final, b_final, nn)


def _blkdiag(W):
    z = jnp.zeros_like(W)
    return jnp.concatenate(
        [jnp.concatenate([W, z], axis=1), jnp.concatenate([z, W], axis=1)],
        axis=0,
    )


def kernel(x, edge_index, iteration, W_init, b_init, W_msg, W_ir, W_iz, W_in,
           W_hr, W_hz, W_hn, b_r, b_z, b_n, W_gate, W_feat, W_final, b_final):
    xf = x.reshape(NF, 26)
    b2 = lambda b: jnp.concatenate([b, b]).reshape(1, 2 * H)
    h0f, tf, uhr, uhz, uhn, sx, dx = _stage1(
        xf, _blkdiag(W_init), b2(b_init), _blkdiag(W_msg), _blkdiag(W_hr),
        _blkdiag(W_hz), _blkdiag(W_hn), edge_index)

    t4 = tf.reshape(T_ROWS, 4, HQ)
    sx = sx.reshape(16, NMB, CPM, CHUNK)
    dx = dx.reshape(16, NMB, CPM, CHUNK)
    mf = _segment_sum(t4, sx, dx).reshape(M_PAD // 2, 2 * H)

    nn = h0f[NF - 1:NF, H:2 * H]
    hgf, comb = _stage3(
        mf, h0f, uhr, uhz, uhn, nn, _blkdiag(W_ir), _blkdiag(W_iz),
        _blkdiag(W_in), b2(b_r), b2(b_z), b2(b_n), _blkdiag(W_gate),
        _blkdiag(W_feat), W_final, b_final.reshape(1, H),
    )
    hg_raw = hgf[:, 0:H] + hgf[:, H:2 * H]
    return jnp.where(iteration != 0, comb, hg_raw).reshape(H)


# speedup vs baseline: 1.0218x; 1.0218x over previous
import jax
import jax.numpy as jnp
from jax import lax
from jax.experimental import pallas as pl
from jax.experimental.pallas import tpu as pltpu
from jax.experimental.pallas import tpu_sc as plsc

N = 50000
E = 800000
H = 64
HQ = 16
NF = N // 2

CHUNK = 512
GK = 1
NG = 10
NMB = 10
CPM = NG * GK
E_PAD = 16 * NMB * NG * GK * CHUNK
BE = E_PAD // 25

T_ROWS = 50176
ACC_ROWS = 50176
ZROWS = 64
M_PAD = 50048
ROWS_PER_SUB = M_PAD // 16
PAD_DST = M_PAD


def _seg_body(t4, sx, dx, m_out, spt, acc, sv, dv, rv0, rv1, zbuf,
              gsem0, gsem1, ssem0, ssem1):
    c = lax.axis_index("c")
    s = lax.axis_index("s")

    zv = jnp.zeros((16,), jnp.float32)

    def _zb(i, carry):
        zbuf[i, pl.ds(0, 16)] = zv
        return carry

    lax.fori_loop(0, ZROWS, _zb, 0)

    rv = (rv0, rv1)
    gsem = (gsem0, gsem1)
    ssem = (ssem0, ssem1)

    def _drain_scatters(B):
        for j in range(GK):
            pltpu.make_async_copy(
                rv[B].at[j], acc.at[dv.at[j]], ssem[B]
            ).wait()

    for q_loc in (0, 1):
        q = 2 * c + q_loc

        def _za(k, carry):
            pltpu.sync_copy(zbuf, acc.at[pl.ds(s * 3136 + k * ZROWS, ZROWS)])
            return carry

        lax.fori_loop(0, 49, _za, 0)
        pltpu.sync_copy(
            t4.at[pl.ds(s * 3136, 3136), q],
            spt.at[pl.ds(s * 3136, 3136)],
        )
        plsc.subcore_barrier()

        def _mega(mb, carry):
            @pl.when(mb > 0)
            def _():
                _drain_scatters(0)
                _drain_scatters(1)

            pltpu.sync_copy(sx.at[s, mb], sv)
            pltpu.sync_copy(dx.at[s, mb], dv)

            def _gpair(gp, inner):
                for B in (0, 1):
                    g = 2 * gp + B

                    @pl.when(g >= 2)
                    def _():
                        _drain_scatters(B)

                    descs = []
                    for j in range(GK):
                        descs.append(pltpu.async_copy(
                            spt.at[sv.at[g * GK + j]], rv[B].at[j], gsem[B]
                        ))
                    for d in descs:
                        d.wait()
                    for j in range(GK):
                        pltpu.async_copy(
                            rv[B].at[j], acc.at[dv.at[g * GK + j]], ssem[B],
                            add=True,
                        )
                return inner

            lax.fori_loop(0, NG // 2, _gpair, 0)
            return carry

        lax.fori_loop(0, NMB, _mega, 0)
        _drain_scatters(0)
        _drain_scatters(1)
        plsc.subcore_barrier()

        pltpu.sync_copy(
            acc.at[pl.ds(s * ROWS_PER_SUB, ROWS_PER_SUB)],
            m_out.at[pl.ds(s * ROWS_PER_SUB, ROWS_PER_SUB), q],
        )
        plsc.subcore_barrier()


def _segment_sum(t4, sx, dx):
    mesh = plsc.VectorSubcoreMesh(
        core_axis_name="c", subcore_axis_name="s", num_cores=2, num_subcores=16
    )
    return pl.kernel(
        _seg_body,
        out_type=jax.ShapeDtypeStruct((M_PAD, 4, HQ), jnp.float32),
        mesh=mesh,
        scratch_types=[
            pltpu.VMEM_SHARED((T_ROWS, HQ), jnp.float32),
            pltpu.VMEM_SHARED((ACC_ROWS, HQ), jnp.float32),
            pltpu.VMEM((CPM, CHUNK), jnp.int32),
            pltpu.VMEM((CPM, CHUNK), jnp.int32),
            pltpu.VMEM((GK, CHUNK, HQ), jnp.float32),
            pltpu.VMEM((GK, CHUNK, HQ), jnp.float32),
            pltpu.VMEM((ZROWS, HQ), jnp.float32),
            pltpu.SemaphoreType.DMA,
            pltpu.SemaphoreType.DMA,
            pltpu.SemaphoreType.DMA,
            pltpu.SemaphoreType.DMA,
        ],
        compiler_params=pltpu.CompilerParams(use_tc_tiling_on_sc=False),
    )(t4, sx, dx)


BRF = 1000


def _s1_body(xf_ref, wi2_ref, bi2_ref, wm2_ref, eix_ref, h0f_ref, tf_ref,
             sx_ref, dx_ref):
    i = pl.program_id(0)
    h0f = jnp.tanh(
        jnp.dot(xf_ref[...], wi2_ref[...], preferred_element_type=jnp.float32)
        + bi2_ref[...]
    )
    h0f_ref[...] = h0f
    tf_ref[...] = jnp.dot(h0f, wm2_ref[...],
                          preferred_element_type=jnp.float32)

    eid = i * BE + lax.broadcasted_iota(jnp.int32, (1, BE), 1)
    valid = eid < E
    src = eix_ref[0:1, :]
    dst = eix_ref[1:2, :]
    sx_ref[...] = jnp.where(valid, src, 0).reshape(1, 1, BE)
    dx_ref[...] = jnp.where(valid, dst, PAD_DST).reshape(1, 1, BE)


def _stage1(xf, W2_init, b2_init, W2_msg, edge_index):
    return pl.pallas_call(
        _s1_body,
        grid=(25,),
        in_specs=[
            pl.BlockSpec((BRF, 26), lambda i: (i, 0)),
            pl.BlockSpec((26, 2 * H), lambda i: (0, 0)),
            pl.BlockSpec((1, 2 * H), lambda i: (0, 0)),
            pl.BlockSpec((2 * H, 2 * H), lambda i: (0, 0)),
            pl.BlockSpec((2, BE), lambda i: (0, i)),
        ],
        out_specs=[
            pl.BlockSpec((BRF, 2 * H), lambda i: (i, 0)),
            pl.BlockSpec((BRF, 2 * H), lambda i: (i, 0)),
            pl.BlockSpec((1, 1, BE), lambda i: (i, 0, 0)),
            pl.BlockSpec((1, 1, BE), lambda i: (i, 0, 0)),
        ],
        out_shape=[
            jax.ShapeDtypeStruct((NF, 2 * H), jnp.float32),
            jax.ShapeDtypeStruct((T_ROWS // 2, 2 * H), jnp.float32),
            jax.ShapeDtypeStruct((25, 1, BE), jnp.int32),
            jax.ShapeDtypeStruct((25, 1, BE), jnp.int32),
        ],
    )(xf, W2_init, b2_init, W2_msg, edge_index)


def _s1b_body(h0f_ref, whr, whz, whn, uhr_ref, uhz_ref, uhn_ref):
    h0f = h0f_ref[...]

    def dot(a, b):
        return jnp.dot(a, b[...], preferred_element_type=jnp.float32)

    uhr_ref[...] = dot(h0f, whr)
    uhz_ref[...] = dot(h0f, whz)
    uhn_ref[...] = dot(h0f, whn)


def _stage1b(h0f, Wd_hr, Wd_hz, Wd_hn):
    full = pl.BlockSpec((2 * H, 2 * H), lambda i: (0, 0))
    rows = pl.BlockSpec((BRF, 2 * H), lambda i: (i, 0))
    out = jax.ShapeDtypeStruct((NF, 2 * H), jnp.float32)
    return pl.pallas_call(
        _s1b_body,
        grid=(25,),
        in_specs=[rows, full, full, full],
        out_specs=[rows, rows, rows],
        out_shape=[out, out, out],
    )(h0f, Wd_hr, Wd_hz, Wd_hn)


def _s3_body(mf_ref, h0f_ref, uhr_ref, uhz_ref, uhn_ref, wir, wiz, win, br,
             bz, bn, wg, wf, wfin, bfin, nn_ref, hg_ref, comb_ref):
    i = pl.program_id(0)
    mf = mf_ref[...]
    h0f = h0f_ref[...]

    def dot(a, b):
        return jnp.dot(a, b[...], preferred_element_type=jnp.float32)

    r = jax.nn.sigmoid(dot(mf, wir) + uhr_ref[...] + br[...])
    z = jax.nn.sigmoid(dot(mf, wiz) + uhz_ref[...] + bz[...])
    cand = jnp.tanh(dot(mf, win) + r * uhn_ref[...] + bn[...])
    h = (1.0 - z) * cand + z * h0f
    gate = jax.nn.sigmoid(dot(h, wg))
    feat = jnp.tanh(dot(h, wf))
    part = jnp.sum(gate * feat, axis=0, keepdims=True)

    @pl.when(i == 0)
    def _():
        hg_ref[...] = jnp.zeros_like(hg_ref)
        comb_ref[...] = jnp.zeros_like(comb_ref)

    hg_ref[...] += part

    @pl.when(i == pl.num_programs(0) - 1)
    def _():
        acc = hg_ref[...]
        hg64 = acc[:, 0:H] + acc[:, H:2 * H]
        cat = jnp.concatenate([hg64, nn_ref[...]], axis=1)
        comb_ref[...] = jnp.tanh(
            jnp.dot(cat, wfin[...], preferred_element_type=jnp.float32)
            + bfin[...]
        )


def _stage3(mf, h0f, uhr, uhz, uhn, nn, Wd_ir, Wd_iz, Wd_in, b2_r, b2_z,
            b2_n, Wd_gate, Wd_feat, W_final, b_final):
    full = pl.BlockSpec((2 * H, 2 * H), lambda i: (0, 0))
    vec2 = pl.BlockSpec((1, 2 * H), lambda i: (0, 0))
    vec = pl.BlockSpec((1, H), lambda i: (0, 0))
    rows = pl.BlockSpec((BRF, 2 * H), lambda i: (i, 0))
    return pl.pallas_call(
        _s3_body,
        grid=(25,),
        in_specs=[rows, rows, rows, rows, rows, full, full, full, vec2, vec2,
                  vec2, full, full, pl.BlockSpec((2 * H, H), lambda i: (0, 0)),
                  vec, vec],
        out_specs=[vec2, vec],
        out_shape=[
            jax.ShapeDtypeStruct((1, 2 * H), jnp.float32),
            jax.ShapeDtypeStruct((1, H), jnp.float32),
        ],
    )(mf, h0f, uhr, uhz, uhn, Wd_ir, Wd_iz, Wd_in, b2_r, b2_z, b2_n,
      Wd_gate, Wd_feat, W_final, b_final, nn)


def _blkdiag(W):
    z = jnp.zeros_like(W)
    return jnp.concatenate(
        [jnp.concatenate([W, z], axis=1), jnp.concatenate([z, W], axis=1)],
        axis=0,
    )


def kernel(x, edge_index, iteration, W_init, b_init, W_msg, W_ir, W_iz, W_in,
           W_hr, W_hz, W_hn, b_r, b_z, b_n, W_gate, W_feat, W_final, b_final):
    xf = x.reshape(NF, 26)
    b2 = lambda b: jnp.concatenate([b, b]).reshape(1, 2 * H)
    h0f, tf, sx, dx = _stage1(xf, _blkdiag(W_init), b2(b_init),
                              _blkdiag(W_msg), edge_index)

    t4 = tf.reshape(T_ROWS, 4, HQ)
    sx = sx.reshape(16, NMB, CPM, CHUNK)
    dx = dx.reshape(16, NMB, CPM, CHUNK)
    mf = _segment_sum(t4, sx, dx).reshape(M_PAD // 2, 2 * H)

    uhr, uhz, uhn = _stage1b(h0f, _blkdiag(W_hr), _blkdiag(W_hz),
                             _blkdiag(W_hn))

    nn = h0f[NF - 1:NF, H:2 * H]
    hgf, comb = _stage3(
        mf, h0f, uhr, uhz, uhn, nn, _blkdiag(W_ir), _blkdiag(W_iz),
        _blkdiag(W_in), b2(b_r), b2(b_z), b2(b_n), _blkdiag(W_gate),
        _blkdiag(W_feat), W_final, b_final.reshape(1, H),
    )
    hg_raw = hgf[:, 0:H] + hgf[:, H:2 * H]
    return jnp.where(iteration != 0, comb, hg_raw).reshape(H)
